# Initial kernel scaffold; baseline (speedup 1.0000x reference)
import jax
import jax.numpy as jnp
from jax.experimental import pallas as pl


def kernel(x, pos_emb_weight):
    raise NotImplementedError("write your pallas kernel here")


# speedup vs baseline: 1.6690x; 1.6690x over previous
import jax
import jax.numpy as jnp
from jax.experimental import pallas as pl

BL = 1024


def _add_kernel(x_ref, pos_ref, o_ref):
    o_ref[...] = x_ref[...] + pos_ref[...]


def kernel(x, pos_emb_weight):
    b, l, d = x.shape
    grid = (l // BL, b)
    return pl.pallas_call(
        _add_kernel,
        grid=grid,
        in_specs=[
            pl.BlockSpec((1, BL, d), lambda i, j: (j, i, 0)),
            pl.BlockSpec((BL, d), lambda i, j: (i, 0)),
        ],
        out_specs=pl.BlockSpec((1, BL, d), lambda i, j: (j, i, 0)),
        out_shape=jax.ShapeDtypeStruct((b, l, d), x.dtype),
    )(x, pos_emb_weight)


# speedup vs baseline: 1.7357x; 1.0399x over previous
import jax
import jax.numpy as jnp
from jax.experimental import pallas as pl

BL = 2048


def _add_kernel(x_ref, pos_ref, o_ref):
    o_ref[...] = x_ref[...] + pos_ref[...]


def kernel(x, pos_emb_weight):
    b, l, d = x.shape
    grid = (l // BL, b)
    return pl.pallas_call(
        _add_kernel,
        grid=grid,
        in_specs=[
            pl.BlockSpec((1, BL, d), lambda i, j: (j, i, 0)),
            pl.BlockSpec((BL, d), lambda i, j: (i, 0)),
        ],
        out_specs=pl.BlockSpec((1, BL, d), lambda i, j: (j, i, 0)),
        out_shape=jax.ShapeDtypeStruct((b, l, d), x.dtype),
    )(x, pos_emb_weight)
